# baseline (speedup 1.0000x reference)
import functools

import jax
import jax.numpy as jnp
from jax.experimental import pallas as pl


def _linear_block(x_ref, ea_ref, w1_ref, w2_ref, b_ref, out_ref):
    acc = jnp.dot(x_ref[...], w1_ref[...], preferred_element_type=jnp.float32)
    acc += jnp.dot(ea_ref[...], w2_ref[...], preferred_element_type=jnp.float32)
    out_ref[...] = acc + b_ref[...]


@functools.partial(jax.jit, static_argnames=())
def kernel(x, pos, edge_index, edge_attr, W, b):
    del pos, edge_index
    n_edges, d_feat = x.shape
    d_edge = edge_attr.shape[1]
    out_ch = W.shape[0]

    w1 = W[:, :d_feat].T
    w2 = W[:, d_feat:].T
    b2 = b.reshape(1, out_ch)

    block = 2000
    grid = (n_edges // block,)

    return pl.pallas_call(
        _linear_block,
        grid=grid,
        in_specs=[
            pl.BlockSpec((block, d_feat), lambda i: (i, 0)),
            pl.BlockSpec((block, d_edge), lambda i: (i, 0)),
            pl.BlockSpec((d_feat, out_ch), lambda i: (0, 0)),
            pl.BlockSpec((d_edge, out_ch), lambda i: (0, 0)),
            pl.BlockSpec((1, out_ch), lambda i: (0, 0)),
        ],
        out_specs=pl.BlockSpec((block, out_ch), lambda i: (i, 0)),
        out_shape=jax.ShapeDtypeStruct((n_edges, out_ch), jnp.float32),
    )(x, edge_attr, w1, w2, b2)


# speedup vs baseline: 1.2434x; 1.2434x over previous
import functools

import jax
import jax.numpy as jnp
from jax.experimental import pallas as pl


def _linear_block(x_ref, ea_ref, w1_ref, w2_ref, b_ref, out_ref):
    acc = jnp.dot(x_ref[...], w1_ref[...], preferred_element_type=jnp.float32)
    acc += jnp.dot(ea_ref[...], w2_ref[...], preferred_element_type=jnp.float32)
    out_ref[...] = acc + b_ref[...]


@functools.partial(jax.jit, static_argnames=())
def kernel(x, pos, edge_index, edge_attr, W, b):
    del pos, edge_index
    n_edges, d_feat = x.shape
    d_edge = edge_attr.shape[1]
    out_ch = W.shape[0]

    w1 = W[:, :d_feat].T
    w2 = W[:, d_feat:].T
    b2 = b.reshape(1, out_ch)

    block = 8000
    grid = (n_edges // block,)

    return pl.pallas_call(
        _linear_block,
        grid=grid,
        in_specs=[
            pl.BlockSpec((block, d_feat), lambda i: (i, 0)),
            pl.BlockSpec((block, d_edge), lambda i: (i, 0)),
            pl.BlockSpec((d_feat, out_ch), lambda i: (0, 0)),
            pl.BlockSpec((d_edge, out_ch), lambda i: (0, 0)),
            pl.BlockSpec((1, out_ch), lambda i: (0, 0)),
        ],
        out_specs=pl.BlockSpec((block, out_ch), lambda i: (i, 0)),
        out_shape=jax.ShapeDtypeStruct((n_edges, out_ch), jnp.float32),
    )(x, edge_attr, w1, w2, b2)


# speedup vs baseline: 1.2510x; 1.0062x over previous
import functools

import jax
import jax.numpy as jnp
from jax.experimental import pallas as pl


def _linear_block(x_ref, ea_ref, w1_ref, w2_ref, b_ref, out_ref):
    acc = jnp.dot(x_ref[...], w1_ref[...], preferred_element_type=jnp.float32)
    acc += jnp.dot(ea_ref[...], w2_ref[...], preferred_element_type=jnp.float32)
    out_ref[...] = acc + b_ref[...]


@functools.partial(jax.jit, static_argnames=())
def kernel(x, pos, edge_index, edge_attr, W, b):
    del pos, edge_index
    n_edges, d_feat = x.shape
    d_edge = edge_attr.shape[1]
    out_ch = W.shape[0]

    w1 = W[:, :d_feat].T
    w2 = W[:, d_feat:].T
    b2 = b.reshape(1, out_ch)

    block = 16000
    grid = (n_edges // block,)

    return pl.pallas_call(
        _linear_block,
        grid=grid,
        in_specs=[
            pl.BlockSpec((block, d_feat), lambda i: (i, 0)),
            pl.BlockSpec((block, d_edge), lambda i: (i, 0)),
            pl.BlockSpec((d_feat, out_ch), lambda i: (0, 0)),
            pl.BlockSpec((d_edge, out_ch), lambda i: (0, 0)),
            pl.BlockSpec((1, out_ch), lambda i: (0, 0)),
        ],
        out_specs=pl.BlockSpec((block, out_ch), lambda i: (i, 0)),
        out_shape=jax.ShapeDtypeStruct((n_edges, out_ch), jnp.float32),
    )(x, edge_attr, w1, w2, b2)


# speedup vs baseline: 1.2539x; 1.0023x over previous
import functools

import jax
import jax.numpy as jnp
from jax.experimental import pallas as pl


def _linear_block(x_ref, ea_ref, w1_ref, w2_ref, b_ref, out_ref):
    xb = x_ref[...].astype(jnp.bfloat16)
    eb = ea_ref[...].astype(jnp.bfloat16)
    acc = jnp.dot(xb, w1_ref[...], preferred_element_type=jnp.float32)
    acc += jnp.dot(eb, w2_ref[...], preferred_element_type=jnp.float32)
    out_ref[...] = acc + b_ref[...]


@functools.partial(jax.jit, static_argnames=())
def kernel(x, pos, edge_index, edge_attr, W, b):
    del pos, edge_index
    n_edges, d_feat = x.shape
    d_edge = edge_attr.shape[1]
    out_ch = W.shape[0]

    w1 = W[:, :d_feat].T.astype(jnp.bfloat16)
    w2 = W[:, d_feat:].T.astype(jnp.bfloat16)
    b2 = b.reshape(1, out_ch)

    block = 16000
    grid = (n_edges // block,)

    return pl.pallas_call(
        _linear_block,
        grid=grid,
        in_specs=[
            pl.BlockSpec((block, d_feat), lambda i: (i, 0)),
            pl.BlockSpec((block, d_edge), lambda i: (i, 0)),
            pl.BlockSpec((d_feat, out_ch), lambda i: (0, 0)),
            pl.BlockSpec((d_edge, out_ch), lambda i: (0, 0)),
            pl.BlockSpec((1, out_ch), lambda i: (0, 0)),
        ],
        out_specs=pl.BlockSpec((block, out_ch), lambda i: (i, 0)),
        out_shape=jax.ShapeDtypeStruct((n_edges, out_ch), jnp.float32),
    )(x, edge_attr, w1, w2, b2)
